# baseline (speedup 1.0000x reference)
import jax
import jax.numpy as jnp
from jax import lax
from jax.experimental import pallas as pl
from jax.experimental.pallas import tpu as pltpu
from jax.experimental.pallas import tpu_sc as plsc

DIM = 128
NT = 16
K = 80
HB = 640
HW = 16


def _s1_body(eidx_hbm, vei_hbm, cei_hbm, ones_hbm, zcol_hbm,
             xv_out, xc_out, deg_out,
             eidx_v, xvb, xcb, ones_v, zcol_v, histv, histc,
             sem_a, sem_b):
  cid = lax.axis_index("c")
  t = lax.axis_index("s")
  ch = eidx_hbm.shape[2]
  pltpu.sync_copy(ones_hbm, ones_v)
  pltpu.sync_copy(zcol_hbm, zcol_v)
  pltpu.sync_copy(zcol_v, histv.at[pl.ds(t * HB, HB)])
  pltpu.sync_copy(zcol_v, histc.at[pl.ds(t * HB, HB)])
  plsc.subcore_barrier()
  pltpu.sync_copy(eidx_hbm.at[cid, t], eidx_v)

  def issue(i, sem):
    pltpu.async_copy(vei_hbm.at[eidx_v.at[i]], xvb.at[i], sem)
    pltpu.async_copy(cei_hbm.at[eidx_v.at[i]], xcb.at[i], sem)

  def drain(i, sem):
    pltpu.make_async_copy(vei_hbm.at[eidx_v.at[i]], xvb.at[i], sem).wait()
    pltpu.make_async_copy(cei_hbm.at[eidx_v.at[i]], xcb.at[i], sem).wait()

  def hist(i):
    pltpu.sync_copy(ones_v, histv.at[xvb.at[i]], add=True)
    pltpu.sync_copy(ones_v, histc.at[xcb.at[i]], add=True)

  issue(0, sem_a)

  def pair(p, carry):
    i = 2 * p
    drain(i, sem_a)
    issue(i + 1, sem_b)
    hist(i)
    drain(i + 1, sem_b)

    @pl.when(i + 2 < ch)
    def _():
      issue(i + 2, sem_a)

    hist(i + 1)
    return carry

  lax.fori_loop(0, ch // 2, pair, 0)
  if ch % 2 == 1:
    drain(ch - 1, sem_a)
    hist(ch - 1)
  pltpu.sync_copy(xvb, xv_out.at[cid, t])
  pltpu.sync_copy(xcb, xc_out.at[cid, t])
  plsc.subcore_barrier()
  pltpu.sync_copy(histv.at[pl.ds(t * HB, HB)], deg_out.at[cid, pl.ds(t * HB, HB)])
  pltpu.sync_copy(histc.at[pl.ds(t * HB, HB)], deg_out.at[cid + 2, pl.ds(t * HB, HB)])


def _edge_setup(p_ei, n_ei, v_ei, c_ei):
  ep = p_ei.shape[0]
  ch = ep // (NT * K)
  hpad = NT * HB
  eidx = jnp.stack([p_ei, n_ei]).reshape(2, NT, ch, K)
  ones = jnp.ones((K, HW), jnp.float32)
  zcol = jnp.zeros((HB, HW), jnp.float32)
  s1 = pl.kernel(
      _s1_body,
      out_type=(
          jax.ShapeDtypeStruct((2, NT, ch, K), jnp.int32),
          jax.ShapeDtypeStruct((2, NT, ch, K), jnp.int32),
          jax.ShapeDtypeStruct((4, hpad, HW), jnp.float32),
      ),
      mesh=plsc.VectorSubcoreMesh(core_axis_name="c", subcore_axis_name="s"),
      compiler_params=pltpu.CompilerParams(use_tc_tiling_on_sc=False),
      scratch_types=[
          pltpu.VMEM((ch, K), jnp.int32),
          pltpu.VMEM((ch, K), jnp.int32),
          pltpu.VMEM((ch, K), jnp.int32),
          pltpu.VMEM((K, HW), jnp.float32),
          pltpu.VMEM((HB, HW), jnp.float32),
          pltpu.VMEM_SHARED((hpad, HW), jnp.float32),
          pltpu.VMEM_SHARED((hpad, HW), jnp.float32),
          pltpu.SemaphoreType.DMA,
          pltpu.SemaphoreType.DMA,
      ],
  )
  return s1(eidx, v_ei, c_ei, ones, zcol)


def _s2_body(m_hbm, xv_hbm, xc_hbm, zacc_hbm, acc_out,
             xvb, xcb, offb, rows_a, rows_b, acc, sem_a, sem_b):
  cid = lax.axis_index("c")
  t = lax.axis_index("s")
  ch = xv_hbm.shape[2]
  n = m_hbm.shape[0] // 4
  n_rows = acc.shape[0]
  rpt = n_rows // NT
  pltpu.sync_copy(xv_hbm.at[cid, t], xvb)
  pltpu.sync_copy(xc_hbm.at[cid, t], xcb)

  def phase(side, gidx, sidx, goff):
    gsrc_hbm = m_hbm
    def mkoff(i, slot):
      for j in range(K // 16):
        offb[slot, pl.ds(j * 16, 16)] = gidx[i, pl.ds(j * 16, 16)] + goff

    pltpu.sync_copy(zacc_hbm, acc.at[pl.ds(t * rpt, rpt)])
    plsc.subcore_barrier()

    mkoff(0, 0)
    pltpu.async_copy(gsrc_hbm.at[offb.at[0]], rows_a, sem_a)

    def pair(p, carry):
      i = 2 * p
      pltpu.make_async_copy(gsrc_hbm.at[offb.at[0]], rows_a, sem_a).wait()
      mkoff(i + 1, 1)
      pltpu.async_copy(gsrc_hbm.at[offb.at[1]], rows_b, sem_b)
      pltpu.sync_copy(rows_a, acc.at[sidx.at[i]], add=True)
      pltpu.make_async_copy(gsrc_hbm.at[offb.at[1]], rows_b, sem_b).wait()

      @pl.when(i + 2 < ch)
      def _():
        mkoff(i + 2, 0)
        pltpu.async_copy(gsrc_hbm.at[offb.at[0]], rows_a, sem_a)

      pltpu.sync_copy(rows_b, acc.at[sidx.at[i + 1]], add=True)
      return carry

    lax.fori_loop(0, ch // 2, pair, 0)
    if ch % 2 == 1:
      pltpu.make_async_copy(gsrc_hbm.at[offb.at[0]], rows_a, sem_a).wait()
      pltpu.sync_copy(rows_a, acc.at[sidx.at[ch - 1]], add=True)
    plsc.subcore_barrier()
    pltpu.sync_copy(acc.at[pl.ds(t * rpt, rpt)],
                    acc_out.at[side, cid, pl.ds(t * rpt, rpt)])
    plsc.subcore_barrier()

  phase(0, xvb, xcb, cid * n)
  phase(1, xcb, xvb, 2 * n + cid * n)


def _aggregate(m, xv, xc, n):
  ch = xv.shape[2]
  npad = NT * ((n + NT * 8 - 1) // (NT * 8)) * 8
  zacc = jnp.zeros((npad // NT, DIM), jnp.bfloat16)
  s2 = pl.kernel(
      _s2_body,
      out_type=jax.ShapeDtypeStruct((2, 2, npad, DIM), jnp.bfloat16),
      mesh=plsc.VectorSubcoreMesh(core_axis_name="c", subcore_axis_name="s"),
      compiler_params=pltpu.CompilerParams(use_tc_tiling_on_sc=False),
      scratch_types=[
          pltpu.VMEM((ch, K), jnp.int32),
          pltpu.VMEM((ch, K), jnp.int32),
          pltpu.VMEM((2, K), jnp.int32),
          pltpu.VMEM((K, DIM), jnp.bfloat16),
          pltpu.VMEM((K, DIM), jnp.bfloat16),
          pltpu.VMEM_SHARED((npad, DIM), jnp.bfloat16),
          pltpu.SemaphoreType.DMA,
          pltpu.SemaphoreType.DMA,
      ],
  )
  return s2(m, xv, xc, zacc)


def _mlp4_body(x_ref, w1_ref, b1_ref, w2_ref, b2_ref, deg_ref, out_ref):
  x = x_ref[0]
  h = jnp.maximum(
      jnp.dot(x, w1_ref[0], preferred_element_type=jnp.float32) + b1_ref[0], 0.0)
  y = jnp.dot(h, w2_ref[0], preferred_element_type=jnp.float32) + b2_ref[0]
  s = 1.0 / jnp.sqrt(jnp.maximum(deg_ref[0, :, 0:1], 1.0))
  out_ref[...] = (y * s).astype(jnp.bfloat16)


def _messages(e, w1s, b1s, w2s, b2s, deg):
  n = e.shape[1]
  b = 1000
  nb = n // b
  return pl.pallas_call(
      _mlp4_body,
      grid=(4, nb),
      in_specs=[
          pl.BlockSpec((1, b, DIM), lambda q, j: (1 - q // 2, j, 0)),
          pl.BlockSpec((1, DIM, DIM), lambda q, j: (q, 0, 0)),
          pl.BlockSpec((1, 1, DIM), lambda q, j: (q, 0, 0)),
          pl.BlockSpec((1, DIM, DIM), lambda q, j: (q, 0, 0)),
          pl.BlockSpec((1, 1, DIM), lambda q, j: (q, 0, 0)),
          pl.BlockSpec((1, b, HW), lambda q, j: (q, j, 0)),
      ],
      out_specs=pl.BlockSpec((b, DIM), lambda q, j: (q * nb + j, 0)),
      out_shape=jax.ShapeDtypeStruct((4 * n, DIM), jnp.bfloat16),
  )(e, w1s, b1s, w2s, b2s, deg)


def _upd_body(x_ref, a_ref, deg_ref, w_ref, b_ref, out_ref):
  sp = 1.0 / jnp.sqrt(jnp.maximum(deg_ref[0, 0, :, 0:1], 1.0))
  sn = 1.0 / jnp.sqrt(jnp.maximum(deg_ref[0, 1, :, 0:1], 1.0))
  w = w_ref[0]
  y = jnp.dot(x_ref[0], w[0:DIM], preferred_element_type=jnp.float32)
  y = y + jnp.dot(a_ref[0, 0].astype(jnp.float32) * sp, w[DIM:2 * DIM],
                  preferred_element_type=jnp.float32)
  y = y + jnp.dot(a_ref[0, 1].astype(jnp.float32) * sn, w[2 * DIM:3 * DIM],
                  preferred_element_type=jnp.float32)
  out_ref[0] = y + b_ref[0]


def _update(e, acc, ddeg, wup, bup):
  n = e.shape[1]
  b = 1000
  nb = n // b
  return pl.pallas_call(
      _upd_body,
      grid=(2, nb),
      in_specs=[
          pl.BlockSpec((1, b, DIM), lambda s, j: (s, j, 0)),
          pl.BlockSpec((1, 2, b, DIM), lambda s, j: (s, 0, j, 0)),
          pl.BlockSpec((1, 2, b, HW), lambda s, j: (1 - s, 0, j, 0)),
          pl.BlockSpec((1, 3 * DIM, DIM), lambda s, j: (s, 0, 0)),
          pl.BlockSpec((1, 1, DIM), lambda s, j: (s, 0, 0)),
      ],
      out_specs=pl.BlockSpec((1, b, DIM), lambda s, j: (s, j, 0)),
      out_shape=jax.ShapeDtypeStruct((2, n, DIM), jnp.float32),
  )(e, acc, ddeg, wup, bup)


def kernel(v_size, c_size, v_edge_index, c_edge_index, p_edge_index,
           n_edge_index, v_emb, c_emb, params):
  n = v_emb.shape[0]
  assert c_emb.shape[0] == n
  xv, xc, deg = _edge_setup(p_edge_index, n_edge_index,
                            v_edge_index, c_edge_index)
  w1s = jnp.stack([params[k]["W1"] for k in ("p_v2c", "n_v2c", "p_c2v", "n_c2v")])
  b1s = jnp.stack([params[k]["b1"] for k in ("p_v2c", "n_v2c", "p_c2v", "n_c2v")]).reshape(4, 1, DIM)
  w2s = jnp.stack([params[k]["W2"] for k in ("p_v2c", "n_v2c", "p_c2v", "n_c2v")])
  b2s = jnp.stack([params[k]["b2"] for k in ("p_v2c", "n_v2c", "p_c2v", "n_c2v")]).reshape(4, 1, DIM)
  wup = jnp.stack([params["c_up_W"], params["v_up_W"]])
  bup = jnp.stack([params["c_up_b"], params["v_up_b"]]).reshape(2, 1, DIM)
  ddeg = deg.reshape(2, 2, deg.shape[1], HW)
  e = jnp.stack([c_emb, v_emb])
  v_embs = [v_emb]
  c_embs = [c_emb]
  for _ in range(2):
    m = _messages(e, w1s, b1s, w2s, b2s, deg)
    acc = _aggregate(m, xv, xc, n)
    e = _update(e, acc, ddeg, wup, bup)
    c_embs.append(e[0])
    v_embs.append(e[1])
  return jnp.stack(v_embs), jnp.stack(c_embs)


# speedup vs baseline: 1.0377x; 1.0377x over previous
import jax
import jax.numpy as jnp
from jax import lax
from jax.experimental import pallas as pl
from jax.experimental.pallas import tpu as pltpu
from jax.experimental.pallas import tpu_sc as plsc

DIM = 128
NT = 16
K = 80
HB = 640
HW = 16


def _s1_body(eidx_hbm, vei_hbm, cei_hbm, ones_hbm, zcol_hbm,
             xv_out, xc_out, degv_out, degc_out,
             eidx_v, xvb, xcb, ones_v, zcol_v, histv, histc,
             sem_a, sem_b):
  cid = lax.axis_index("c")
  t = lax.axis_index("s")
  ch = eidx_hbm.shape[2]
  pltpu.sync_copy(ones_hbm, ones_v)
  pltpu.sync_copy(zcol_hbm, zcol_v)
  pltpu.sync_copy(zcol_v, histv.at[pl.ds(t * HB, HB)])
  pltpu.sync_copy(zcol_v, histc.at[pl.ds(t * HB, HB)])
  plsc.subcore_barrier()
  pltpu.sync_copy(eidx_hbm.at[cid, t], eidx_v)

  def issue(i, sem):
    pltpu.async_copy(vei_hbm.at[eidx_v.at[i]], xvb.at[i], sem)
    pltpu.async_copy(cei_hbm.at[eidx_v.at[i]], xcb.at[i], sem)

  def drain(i, sem):
    pltpu.make_async_copy(vei_hbm.at[eidx_v.at[i]], xvb.at[i], sem).wait()
    pltpu.make_async_copy(cei_hbm.at[eidx_v.at[i]], xcb.at[i], sem).wait()

  def hist(i):
    pltpu.sync_copy(ones_v, histv.at[xvb.at[i]], add=True)
    pltpu.sync_copy(ones_v, histc.at[xcb.at[i]], add=True)

  issue(0, sem_a)

  def pair(p, carry):
    i = 2 * p
    drain(i, sem_a)
    issue(i + 1, sem_b)
    hist(i)
    drain(i + 1, sem_b)

    @pl.when(i + 2 < ch)
    def _():
      issue(i + 2, sem_a)

    hist(i + 1)
    return carry

  lax.fori_loop(0, ch // 2, pair, 0)
  if ch % 2 == 1:
    drain(ch - 1, sem_a)
    hist(ch - 1)
  pltpu.sync_copy(xvb, xv_out.at[cid, t])
  pltpu.sync_copy(xcb, xc_out.at[cid, t])
  plsc.subcore_barrier()
  pltpu.sync_copy(histv.at[pl.ds(t * HB, HB)], degv_out.at[cid, pl.ds(t * HB, HB)])
  pltpu.sync_copy(histc.at[pl.ds(t * HB, HB)], degc_out.at[cid, pl.ds(t * HB, HB)])


def _edge_setup(p_ei, n_ei, v_ei, c_ei, nv, nc):
  ep = p_ei.shape[0]
  k1 = 125
  ch1 = ep // (NT * k1)
  hpad = NT * HB
  eidx = jnp.stack([p_ei, n_ei]).reshape(2, NT, ch1, k1)
  ones = jnp.ones((k1, HW), jnp.float32)
  zcol = jnp.zeros((HB, HW), jnp.float32)
  s1 = pl.kernel(
      _s1_body,
      out_type=(
          jax.ShapeDtypeStruct((2, NT, ch1, k1), jnp.int32),
          jax.ShapeDtypeStruct((2, NT, ch1, k1), jnp.int32),
          jax.ShapeDtypeStruct((2, hpad, HW), jnp.float32),
          jax.ShapeDtypeStruct((2, hpad, HW), jnp.float32),
      ),
      mesh=plsc.VectorSubcoreMesh(core_axis_name="c", subcore_axis_name="s"),
      compiler_params=pltpu.CompilerParams(use_tc_tiling_on_sc=False),
      scratch_types=[
          pltpu.VMEM((ch1, k1), jnp.int32),
          pltpu.VMEM((ch1, k1), jnp.int32),
          pltpu.VMEM((ch1, k1), jnp.int32),
          pltpu.VMEM((k1, HW), jnp.float32),
          pltpu.VMEM((HB, HW), jnp.float32),
          pltpu.VMEM_SHARED((hpad, HW), jnp.float32),
          pltpu.VMEM_SHARED((hpad, HW), jnp.float32),
          pltpu.SemaphoreType.DMA,
          pltpu.SemaphoreType.DMA,
      ],
  )
  xv, xc, degv, degc = s1(eidx, v_ei, c_ei, ones, zcol)
  ch = ep // (NT * K)
  xv = xv.reshape(2, NT, ch, K)
  xc = xc.reshape(2, NT, ch, K)
  return xv, xc, degv[:, :nv, 0:1], degc[:, :nc, 0:1]


def _s2_body(mv_hbm, mc_hbm, xv_hbm, xc_hbm, zacc_hbm,
             accc_out, accv_out,
             xvb, xcb, offb, rows_a, rows_b, acc, sem_a, sem_b):
  cid = lax.axis_index("c")
  t = lax.axis_index("s")
  ch = xv_hbm.shape[2]
  n_rows = acc.shape[0]
  rpt = n_rows // NT
  pltpu.sync_copy(xv_hbm.at[cid, t], xvb)
  pltpu.sync_copy(xc_hbm.at[cid, t], xcb)

  def phase(gsrc_hbm, gidx, sidx, out_hbm, goff):
    def mkoff(i, slot):
      for j in range(K // 16):
        offb[slot, pl.ds(j * 16, 16)] = gidx[i, pl.ds(j * 16, 16)] + goff

    pltpu.sync_copy(zacc_hbm, acc.at[pl.ds(t * rpt, rpt)])
    plsc.subcore_barrier()

    mkoff(0, 0)
    pltpu.async_copy(gsrc_hbm.at[offb.at[0]], rows_a, sem_a)

    def pair(p, carry):
      i = 2 * p
      pltpu.make_async_copy(gsrc_hbm.at[offb.at[0]], rows_a, sem_a).wait()
      mkoff(i + 1, 1)
      pltpu.async_copy(gsrc_hbm.at[offb.at[1]], rows_b, sem_b)
      pltpu.sync_copy(rows_a, acc.at[sidx.at[i]], add=True)
      pltpu.make_async_copy(gsrc_hbm.at[offb.at[1]], rows_b, sem_b).wait()

      @pl.when(i + 2 < ch)
      def _():
        mkoff(i + 2, 0)
        pltpu.async_copy(gsrc_hbm.at[offb.at[0]], rows_a, sem_a)

      pltpu.sync_copy(rows_b, acc.at[sidx.at[i + 1]], add=True)
      return carry

    lax.fori_loop(0, ch // 2, pair, 0)
    if ch % 2 == 1:
      pltpu.make_async_copy(gsrc_hbm.at[offb.at[0]], rows_a, sem_a).wait()
      pltpu.sync_copy(rows_a, acc.at[sidx.at[ch - 1]], add=True)
    plsc.subcore_barrier()
    pltpu.sync_copy(acc.at[pl.ds(t * rpt, rpt)],
                    out_hbm.at[cid, pl.ds(t * rpt, rpt)])
    plsc.subcore_barrier()

  phase(mv_hbm, xvb, xcb, accc_out, cid * (mv_hbm.shape[0] // 2))
  phase(mc_hbm, xcb, xvb, accv_out, cid * (mc_hbm.shape[0] // 2))


def _aggregate(mv, mc, xv, xc, nv, nc):
  ch = xv.shape[2]
  npad = NT * ((max(nv, nc) + NT * 8 - 1) // (NT * 8)) * 8
  zacc = jnp.zeros((npad // NT, DIM), jnp.bfloat16)
  s2 = pl.kernel(
      _s2_body,
      out_type=(
          jax.ShapeDtypeStruct((2, npad, DIM), jnp.bfloat16),
          jax.ShapeDtypeStruct((2, npad, DIM), jnp.bfloat16),
      ),
      mesh=plsc.VectorSubcoreMesh(core_axis_name="c", subcore_axis_name="s"),
      compiler_params=pltpu.CompilerParams(use_tc_tiling_on_sc=False),
      scratch_types=[
          pltpu.VMEM((ch, K), jnp.int32),
          pltpu.VMEM((ch, K), jnp.int32),
          pltpu.VMEM((2, K), jnp.int32),
          pltpu.VMEM((K, DIM), jnp.bfloat16),
          pltpu.VMEM((K, DIM), jnp.bfloat16),
          pltpu.VMEM_SHARED((npad, DIM), jnp.bfloat16),
          pltpu.SemaphoreType.DMA,
          pltpu.SemaphoreType.DMA,
      ],
  )
  acc_c, acc_v = s2(mv, mc, xv, xc, zacc)
  return acc_c[:, :nc], acc_v[:, :nv]


def _mlp2_body(x_ref, w1_ref, b1_ref, w2_ref, b2_ref, deg_ref, out_ref):
  x = x_ref[...]
  h = jnp.maximum(
      jnp.dot(x, w1_ref[0], preferred_element_type=jnp.float32) + b1_ref[0], 0.0)
  y = jnp.dot(h, w2_ref[0], preferred_element_type=jnp.float32) + b2_ref[0]
  s = 1.0 / jnp.sqrt(jnp.maximum(deg_ref[0], 1.0))
  out_ref[0] = (y * s).astype(jnp.bfloat16)


def _messages(x, wp, wn, degs):
  n = x.shape[0]
  b = 1000
  nb = n // b
  w1s = jnp.stack([wp["W1"], wn["W1"]])
  b1s = jnp.stack([wp["b1"], wn["b1"]]).reshape(2, 1, DIM)
  w2s = jnp.stack([wp["W2"], wn["W2"]])
  b2s = jnp.stack([wp["b2"], wn["b2"]]).reshape(2, 1, DIM)
  out = pl.pallas_call(
      _mlp2_body,
      grid=(2, nb),
      in_specs=[
          pl.BlockSpec((b, DIM), lambda p, j: (j, 0)),
          pl.BlockSpec((1, DIM, DIM), lambda p, j: (p, 0, 0)),
          pl.BlockSpec((1, 1, DIM), lambda p, j: (p, 0, 0)),
          pl.BlockSpec((1, DIM, DIM), lambda p, j: (p, 0, 0)),
          pl.BlockSpec((1, 1, DIM), lambda p, j: (p, 0, 0)),
          pl.BlockSpec((1, b, 1), lambda p, j: (p, j, 0)),
      ],
      out_specs=pl.BlockSpec((1, b, DIM), lambda p, j: (p, j, 0)),
      out_shape=jax.ShapeDtypeStruct((2, n, DIM), jnp.bfloat16),
  )(x, w1s, b1s, w2s, b2s, degs)
  return out.reshape(2 * n, DIM)


def _upd_body(x_ref, a_ref, deg_ref, w_ref, b_ref, out_ref):
  sp = 1.0 / jnp.sqrt(jnp.maximum(deg_ref[0], 1.0))
  sn = 1.0 / jnp.sqrt(jnp.maximum(deg_ref[1], 1.0))
  w = w_ref[...]
  y = jnp.dot(x_ref[...], w[0:DIM], preferred_element_type=jnp.float32)
  y = y + jnp.dot(a_ref[0].astype(jnp.float32) * sp, w[DIM:2 * DIM],
                  preferred_element_type=jnp.float32)
  y = y + jnp.dot(a_ref[1].astype(jnp.float32) * sn, w[2 * DIM:3 * DIM],
                  preferred_element_type=jnp.float32)
  out_ref[...] = y + b_ref[...]


def _update(x, a, degs, w, bias):
  n = x.shape[0]
  b = 1000
  nb = n // b
  return pl.pallas_call(
      _upd_body,
      grid=(nb,),
      in_specs=[
          pl.BlockSpec((b, DIM), lambda j: (j, 0)),
          pl.BlockSpec((2, b, DIM), lambda j: (0, j, 0)),
          pl.BlockSpec((2, b, 1), lambda j: (0, j, 0)),
          pl.BlockSpec((3 * DIM, DIM), lambda j: (0, 0)),
          pl.BlockSpec((1, DIM), lambda j: (0, 0)),
      ],
      out_specs=pl.BlockSpec((b, DIM), lambda j: (j, 0)),
      out_shape=jax.ShapeDtypeStruct((n, DIM), jnp.float32),
  )(x, a, degs, w, bias.reshape(1, DIM))


def kernel(v_size, c_size, v_edge_index, c_edge_index, p_edge_index,
           n_edge_index, v_emb, c_emb, params):
  nv = v_emb.shape[0]
  nc = c_emb.shape[0]
  xv, xc, degv, degc = _edge_setup(p_edge_index, n_edge_index,
                                   v_edge_index, c_edge_index, nv, nc)
  v_embs = [v_emb]
  c_embs = [c_emb]
  for _ in range(2):
    mv = _messages(v_emb, params["p_v2c"], params["n_v2c"], degv)
    mc = _messages(c_emb, params["p_c2v"], params["n_c2v"], degc)
    acc_c, acc_v = _aggregate(mv, mc, xv, xc, nv, nc)
    c_emb = _update(c_emb, acc_c, degc, params["c_up_W"], params["c_up_b"])
    c_embs.append(c_emb)
    v_emb = _update(v_emb, acc_v, degv, params["v_up_W"], params["v_up_b"])
    v_embs.append(v_emb)
  return jnp.stack(v_embs), jnp.stack(c_embs)
